# Initial kernel scaffold; baseline (speedup 1.0000x reference)
import jax
import jax.numpy as jnp
from jax.experimental import pallas as pl


def kernel(x, edge_attr, dist, edge_index, triplet_index, params):
    raise NotImplementedError("write your pallas kernel here")


# speedup vs baseline: 2.1021x; 2.1021x over previous
import functools

import jax
import jax.numpy as jnp
from jax import lax
from jax.experimental import pallas as pl
from jax.experimental.pallas import tpu as pltpu
from jax.experimental.pallas import tpu_sc as plsc

N = 10000
E = 160000
T = 160000
H = 128
DE = 16
NRBF = 16
NSBF = 7
NB = 8
NL = 4
NOUT = 3

NC = 2
NS = 16
NW = NC * NS


def _sc_gather(table, idx, chunk):
    B = idx.shape[0]
    R, D = table.shape
    per_w = B // NW
    n_chunks = per_w // chunk
    assert per_w % chunk == 0 and chunk % 8 == 0 and B % NW == 0
    assert n_chunks % 2 == 1 and n_chunks >= 3

    mesh = plsc.VectorSubcoreMesh(core_axis_name="c", subcore_axis_name="s")

    @functools.partial(
        pl.kernel,
        mesh=mesh,
        out_type=jax.ShapeDtypeStruct((B, D), jnp.float32),
        scratch_types=[
            pltpu.VMEM((per_w,), jnp.int32),
            pltpu.VMEM((chunk, D), jnp.float32),
            pltpu.VMEM((chunk, D), jnp.float32),
            pltpu.SemaphoreType.DMA,
            pltpu.SemaphoreType.DMA,
        ],
        name=f"sc_gather_{R}x{D}_{B}",
    )
    def k(table_hbm, idx_hbm, out_hbm, idx_v, rows0, rows1, sem0, sem1):
        wid = lax.axis_index("s") * NC + lax.axis_index("c")
        base = wid * per_w
        pltpu.sync_copy(idx_hbm.at[pl.ds(base, per_w)], idx_v)

        def start(ci, rows, sem):
            pltpu.async_copy(
                table_hbm.at[idx_v.at[pl.ds(ci * chunk, chunk)]], rows, sem)

        def fin(ci, rows, sem):
            pltpu.make_async_copy(
                table_hbm.at[idx_v.at[pl.ds(0, chunk)]], rows, sem).wait()
            pltpu.sync_copy(rows, out_hbm.at[pl.ds(base + ci * chunk, chunk)])

        start(0, rows0, sem0)
        start(1, rows1, sem1)

        def body(i2, _):
            c0 = 2 * i2
            fin(c0, rows0, sem0)

            @pl.when(c0 + 2 < n_chunks)
            def _():
                start(c0 + 2, rows0, sem0)

            fin(c0 + 1, rows1, sem1)

            @pl.when(c0 + 3 < n_chunks)
            def _():
                start(c0 + 3, rows1, sem1)

            return 0

        lax.fori_loop(0, n_chunks // 2, body, 0)
        fin(n_chunks - 1, rows0, sem0)

    return k(table, idx)


def _sc_scatter_add(values, idx, R, phases):
    B, D = values.shape
    assert D == H
    per_w = B // NS
    r_half = R // NC
    assert sum(phases) == r_half
    SLAB = 80
    n_slab = per_w // SLAB
    assert per_w % SLAB == 0 and n_slab % 2 == 1 and n_slab >= 3
    max_phase = max(phases)
    ZB = 16
    n_zchunk_max = (max_phase + 80 + ZB - 1) // ZB
    acc_rows = n_zchunk_max * ZB
    ZR = 200
    assert all(p % ZR == 0 for p in phases)

    mesh = plsc.VectorSubcoreMesh(core_axis_name="c", subcore_axis_name="s")

    @functools.partial(
        pl.kernel,
        mesh=mesh,
        out_type=jax.ShapeDtypeStruct((R, D), jnp.float32),
        scratch_types=[
            pltpu.VMEM((SLAB, H), jnp.float32),
            pltpu.VMEM((SLAB, H), jnp.float32),
            pltpu.VMEM((SLAB,), jnp.int32),
            pltpu.VMEM((SLAB,), jnp.int32),
            pltpu.VMEM((1, SLAB), jnp.int32),
            pltpu.VMEM((1, SLAB), jnp.int32),
            pltpu.VMEM((ZB, H), jnp.float32),
            pltpu.VMEM((16,), jnp.int32),
            pltpu.VMEM_SHARED((acc_rows, H), jnp.float32),
            pltpu.SemaphoreType.DMA,
            pltpu.SemaphoreType.DMA,
            pltpu.SemaphoreType.DMA,
            pltpu.SemaphoreType.DMA,
        ],
        name=f"sc_scatter_{B}to{R}",
    )
    def k(vals_hbm, idx_hbm, lov_hbm, out_hbm,
          rows0, rows1, isl0, isl1, lidx0, lidx1, zero_v, lo_v, acc_sh,
          sem0, sem1, semz, sems):
        c = lax.axis_index("c")
        s = lax.axis_index("s")
        base = s * per_w
        pltpu.sync_copy(lov_hbm.at[c], lo_v)
        lo_c = lo_v[...]
        zf = jnp.zeros((16,), jnp.float32)

        iota16 = lax.iota(jnp.int32, 16)

        def zinit(i, _):
            r = i // 8
            q = i - r * 8
            zero_v[r, pl.ds(q * 16, 16)] = zf
            return 0

        lax.fori_loop(0, ZB * 8, zinit, 0)

        def start(sl, rows, isl, sem):
            pltpu.async_copy(
                vals_hbm.at[pl.ds(base + sl * SLAB, SLAB)], rows, sem)
            pltpu.async_copy(
                idx_hbm.at[pl.ds(base + sl * SLAB, SLAB)], isl, sem)

        p_lo = 0
        for p, phase_rows in enumerate(phases):
            hrow = jnp.full((16,), phase_rows, jnp.int32)
            lov = lo_c + jnp.full((16,), p_lo, jnp.int32)
            hiv = lov + hrow
            n_z = (phase_rows + 80 + ZB - 1) // ZB

            n_zw = (n_z + NS - 1) // NS
            for j in range(n_zw):
                zi = j * NS + s

                @pl.when(zi < n_z)
                def _():
                    pltpu.async_copy(zero_v, acc_sh.at[pl.ds(zi * ZB, ZB)],
                                     semz)

            for j in range(n_zw):
                zi = j * NS + s

                @pl.when(zi < n_z)
                def _():
                    pltpu.make_async_copy(
                        zero_v, acc_sh.at[pl.ds(0, ZB)], semz).wait()

            plsc.subcore_barrier()

            def fin(sl, rows, isl, lidx, sem):
                pltpu.make_async_copy(
                    vals_hbm.at[pl.ds(0, SLAB)], rows, sem).wait()
                pltpu.make_async_copy(
                    idx_hbm.at[pl.ds(0, SLAB)], isl, sem).wait()
                for q in range(SLAB // 16):
                    trash = hrow + jnp.full((16,), q * 16, jnp.int32) + iota16
                    v = isl[pl.ds(q * 16, 16)]
                    loc = lax.select(v >= lov, v - lov, trash)
                    loc = lax.select(v < hiv, loc, trash)
                    lidx[0, pl.ds(q * 16, 16)] = loc
                pltpu.sync_copy(rows, acc_sh.at[lidx.at[0]], add=True)

            start(0, rows0, isl0, sem0)
            start(1, rows1, isl1, sem1)

            def sbody(i2, _):
                sl0 = 2 * i2
                fin(sl0, rows0, isl0, lidx0, sem0)

                @pl.when(sl0 + 2 < n_slab)
                def _():
                    start(sl0 + 2, rows0, isl0, sem0)

                fin(sl0 + 1, rows1, isl1, lidx1, sem1)

                @pl.when(sl0 + 3 < n_slab)
                def _():
                    start(sl0 + 3, rows1, isl1, sem1)

                return 0

            lax.fori_loop(0, n_slab // 2, sbody, 0)
            fin(n_slab - 1, rows0, isl0, lidx0, sem0)
            plsc.subcore_barrier()

            n_wb = phase_rows // ZR
            n_ww = (n_wb + NS - 1) // NS
            for j in range(n_ww):
                zi = j * NS + s

                @pl.when(zi < n_wb)
                def _():
                    pltpu.async_copy(acc_sh.at[pl.ds(zi * ZR, ZR)],
                                     out_hbm.at[pl.ds(c * r_half + p_lo
                                                      + zi * ZR, ZR)], semz)

            for j in range(n_ww):
                zi = j * NS + s

                @pl.when(zi < n_wb)
                def _():
                    pltpu.make_async_copy(
                        acc_sh.at[pl.ds(0, ZR)],
                        out_hbm.at[pl.ds(0, ZR)], semz).wait()

            plsc.subcore_barrier()
            p_lo += phase_rows

    lov_all = (jnp.arange(NC, dtype=jnp.int32) * r_half)[:, None] + jnp.zeros(
        (NC, 16), jnp.int32)
    return k(values, idx, lov_all)


BE = 1600
BT = 1600


def _sig(v):
    return 1.0 / (1.0 + jnp.exp(-v))


def _sw(v):
    return v * _sig(v)


def _mm(a, w):
    return jnp.dot(a, w, preferred_element_type=jnp.float32)


def _res_blk(v, w1, b1, w2, b2):
    return v + _mm(_sw(_mm(_sw(v), w1) + b1), w2) + b2


def _sin_poly(x):
    x2 = x * x
    return x * (1.0 + x2 * (-1.0 / 6.0 + x2 * (1.0 / 120.0 + x2 * (-1.0 / 5040.0))))


def _cos_poly(x):
    x2 = x * x
    return 1.0 + x2 * (-0.5 + x2 * (1.0 / 24.0 + x2 * (-1.0 / 720.0 + x2 * (1.0 / 40320.0))))


def _cheb_sin_table(s1, co, n):
    cols = []
    um1 = jnp.ones_like(co)
    u = 2.0 * co
    cols.append(s1 * um1)
    for _ in range(n - 1):
        cols.append(s1 * u)
        um1, u = u, 2.0 * co * u - um1
    return cols


def _full_spec(shape):
    return pl.BlockSpec(shape, lambda i: tuple(0 for _ in shape))


def _row_spec(bs, d):
    return pl.BlockSpec((bs, d), lambda i: (i, 0))


def _tc_node_embed(x, wn, bn):
    BN = 400

    def body(x_r, w_r, b_r, o_r):
        o_r[...] = _mm(x_r[...], w_r[...]) + b_r[...]

    return pl.pallas_call(
        body,
        grid=(N // BN,),
        in_specs=[_row_spec(BN, H), _full_spec((H, H)), _full_spec((1, H))],
        out_specs=_row_spec(BN, H),
        out_shape=jax.ShapeDtypeStruct((N, H), jnp.float32),
    )(x, wn, bn.reshape(1, H))


def _tc_embed(xi_g, xj_g, ea, dist2d, wts):
    def body(xi_r, xj_r, ea_r, d_r,
             we_r, be_r, wr_r, br_r, w3_r, b3_r,
             wkj_r, bkj_r, wrbf_r,
             m_o, rbf_o, u_o, xkf_o):
        d = d_r[...]
        t0 = d * (jnp.pi / 5.0)
        s1 = _sin_poly(t0)
        co = _cos_poly(t0)
        dinv = 1.0 / (d + 1e-8)
        cols = _cheb_sin_table(s1, co, NRBF)
        rbf = jnp.concatenate(cols, axis=1) * dinv
        rbf_o[...] = rbf

        ea_v = ea_r[...]
        nrm = jnp.sqrt(jnp.sum(ea_v * ea_v, axis=1, keepdims=True))
        u = ea_v / jnp.maximum(nrm, 1e-8)
        u_o[...] = jnp.concatenate(
            [u, jnp.zeros((u.shape[0], H - DE), jnp.float32)], axis=1)

        e_lin = _mm(ea_v, we_r[...]) + be_r[...]
        r_lin = _mm(rbf, wr_r[...]) + br_r[...]
        xixj = xi_r[...] * xj_r[...]
        w3 = w3_r[...]
        m = _sw(_mm(xixj, w3[0]) + _mm(e_lin, w3[1]) + _mm(r_lin, w3[2])
                + b3_r[...])
        m_o[...] = m

        xkf_o[...] = _sw(_mm(m, wkj_r[...]) + bkj_r[...]) * _mm(rbf, wrbf_r[...])

    (we, be, wr, br, w3, b3, wkj, bkj, wrbf) = wts
    return pl.pallas_call(
        body,
        grid=(E // BE,),
        in_specs=[
            _row_spec(BE, H), _row_spec(BE, H), _row_spec(BE, DE),
            _row_spec(BE, 1),
            _full_spec((DE, H)), _full_spec((1, H)),
            _full_spec((NRBF, H)), _full_spec((1, H)),
            _full_spec((3, H, H)), _full_spec((1, H)),
            _full_spec((H, H)), _full_spec((1, H)),
            _full_spec((NRBF, H)),
        ],
        out_specs=[
            _row_spec(BE, H), _row_spec(BE, NRBF), _row_spec(BE, H),
            _row_spec(BE, H),
        ],
        out_shape=[
            jax.ShapeDtypeStruct((E, H), jnp.float32),
            jax.ShapeDtypeStruct((E, NRBF), jnp.float32),
            jax.ShapeDtypeStruct((E, H), jnp.float32),
            jax.ShapeDtypeStruct((E, H), jnp.float32),
        ],
    )(xi_g, xj_g, ea, dist2d, we, be, wr, br, w3, b3, wkj, bkj, wrbf)


def _tc_sbf(ukj_g, uji_g):
    def body(a_r, b_r, o_r):
        cos = jnp.sum(a_r[:, :DE] * b_r[:, :DE], axis=1, keepdims=True)
        c = jnp.clip(cos, -0.999, 0.999)
        s1 = jnp.sqrt(1.0 - c * c)
        cols = _cheb_sin_table(s1, c, NSBF)
        cols.append(jnp.zeros_like(c))
        o_r[...] = jnp.concatenate(cols, axis=1)

    return pl.pallas_call(
        body,
        grid=(T // BT,),
        in_specs=[_row_spec(BT, H), _row_spec(BT, H)],
        out_specs=_row_spec(BT, NB),
        out_shape=jax.ShapeDtypeStruct((T, NB), jnp.float32),
    )(ukj_g, uji_g)


def _tc_outblock(m, s_in, ow, ob):
    first = s_in is None

    def body(*refs):
        if first:
            m_r, ow_r, ob_r, s_o = refs
        else:
            m_r, s_r, ow_r, ob_r, s_o = refs
        ow_v = ow_r[...]
        ob_v = ob_r[...]
        v = m_r[...]
        for i in range(NOUT):
            v = _res_blk(v, ow_v[2 * i], ob_v[2 * i:2 * i + 1, 0],
                         ow_v[2 * i + 1], ob_v[2 * i + 1:2 * i + 2, 0])
        v = _sw(_mm(v, ow_v[6]) + ob_v[6:7, 0])
        s_o[...] = v if first else s_r[...] + v

    in_specs = [_row_spec(BE, H)]
    args = [m]
    if not first:
        in_specs.append(_row_spec(BE, H))
        args.append(s_in)
    in_specs += [_full_spec((7, H, H)), _full_spec((7, 1, H))]
    args += [ow, ob]
    return pl.pallas_call(
        body,
        grid=(E // BE,),
        in_specs=in_specs,
        out_specs=_row_spec(BE, H),
        out_shape=jax.ShapeDtypeStruct((E, H), jnp.float32),
    )(*args)


def _tc_triplet(xk_g, sbf, wflat, lsb):
    def body(xk_r, sbf_r, wf_r, ls_r, o_r):
        sp = _mm(sbf_r[...], ls_r[...])
        xk = xk_r[...].astype(jnp.bfloat16)
        wf = wf_r[...]
        acc = sp[:, 0:1] * _mm(xk, wf[0])
        for b in range(1, NB):
            acc = acc + sp[:, b:b + 1] * _mm(xk, wf[b])
        o_r[...] = acc

    return pl.pallas_call(
        body,
        grid=(T // BT,),
        in_specs=[
            _row_spec(BT, H), _row_spec(BT, NB),
            _full_spec((NB, H, H)), _full_spec((NB, NB)),
        ],
        out_specs=_row_spec(BT, H),
        out_shape=jax.ShapeDtypeStruct((T, H), jnp.float32),
    )(xk_g, sbf, wflat, lsb)


def _tc_edge(m, aggr, rbf, wts, last):
    def body(m_r, ag_r, rbf_r, ws_r, bs_r, wrbf_r, *out_refs):
        ws = ws_r[...]
        bs = bs_r[...]
        m_v = m_r[...]
        mm = _sw(_mm(m_v, ws[0]) + bs[0:1, 0]) + ag_r[...]
        mm = _res_blk(mm, ws[1], bs[1:2, 0], ws[2], bs[2:3, 0])
        mm = _res_blk(mm, ws[3], bs[3:4, 0], ws[4], bs[4:5, 0])
        m_new = m_v + _sw(_mm(mm, ws[5]) + bs[5:6, 0])

        if last:
            m_o, = out_refs
            m_o[...] = m_new
        else:
            m_o, xkf_o = out_refs
            m_o[...] = m_new
            xkf_o[...] = (_sw(_mm(m_new, ws[6]) + bs[6:7, 0])
                          * _mm(rbf_r[...], wrbf_r[...]))

    ws, bs, wrbf = wts
    nw = ws.shape[0]
    nout_arrs = 1 if last else 2
    return pl.pallas_call(
        body,
        grid=(E // BE,),
        in_specs=[
            _row_spec(BE, H), _row_spec(BE, H), _row_spec(BE, NRBF),
            _full_spec((nw, H, H)), _full_spec((nw, 1, H)),
            _full_spec((NRBF, H)),
        ],
        out_specs=[_row_spec(BE, H)] * nout_arrs,
        out_shape=[jax.ShapeDtypeStruct((E, H), jnp.float32)] * nout_arrs,
    )(m, aggr, rbf, ws, bs, wrbf)


def kernel(x, edge_attr, dist, edge_index, triplet_index, params):
    emb = params["emb"]
    ints = params["int"]
    outs = params["out"]

    i_idx = edge_index[0].astype(jnp.int32)
    j_idx = edge_index[1].astype(jnp.int32)
    kj = triplet_index[0].astype(jnp.int32)
    ji = triplet_index[1].astype(jnp.int32)

    xemb = _tc_node_embed(x, emb["node"]["w"], emb["node"]["b"])
    xi_g = _sc_gather(xemb, i_idx, 200)
    xj_g = _sc_gather(xemb, j_idx, 200)

    def stack_out(p):
        ws = []
        bs = []
        for rp in p["res"]:
            ws += [rp["lin1"]["w"], rp["lin2"]["w"]]
            bs += [rp["lin1"]["b"], rp["lin2"]["b"]]
        ws.append(p["lin"]["w"])
        bs.append(p["lin"]["b"])
        return jnp.stack(ws), jnp.stack(bs)[:, None, :]

    l1 = ints[0]
    wts0 = (emb["edge"]["w"], emb["edge"]["b"].reshape(1, H),
            emb["rbf"]["w"], emb["rbf"]["b"].reshape(1, H),
            jnp.stack([emb["lin"]["w"][0:H], emb["lin"]["w"][H:2 * H],
                       emb["lin"]["w"][2 * H:3 * H]]),
            emb["lin"]["b"].reshape(1, H),
            l1["lin_kj"]["w"], l1["lin_kj"]["b"].reshape(1, H),
            l1["lin_rbf"])
    m, rbf, u, xkf = _tc_embed(
        xi_g, xj_g, edge_attr, dist.reshape(E, 1), wts0)
    ow0, ob0 = stack_out(outs[0])
    s_acc = _tc_outblock(m, None, ow0, ob0)

    ukj_g = _sc_gather(u, kj, 200)
    uji_g = _sc_gather(u, ji, 200)
    sbf = _tc_sbf(ukj_g, uji_g)

    for l in range(NL):
        p = ints[l]
        last = l == NL - 1

        xk_g = _sc_gather(xkf, kj, 200)
        wflat = jnp.transpose(p["W"], (1, 0, 2)).astype(jnp.bfloat16)
        lsb = jnp.concatenate(
            [p["lin_sbf"], jnp.zeros((1, NB), jnp.float32)], axis=0)
        msg = _tc_triplet(xk_g, sbf, wflat, lsb)
        aggr = _sc_scatter_add(msg, ji, E, [13400] * 5 + [13000])

        ws = [p["lin_ji"]["w"],
              p["res1"]["lin1"]["w"], p["res1"]["lin2"]["w"],
              p["res2"]["lin1"]["w"], p["res2"]["lin2"]["w"],
              p["lin_out"]["w"]]
        bs = [p["lin_ji"]["b"],
              p["res1"]["lin1"]["b"], p["res1"]["lin2"]["b"],
              p["res2"]["lin1"]["b"], p["res2"]["lin2"]["b"],
              p["lin_out"]["b"]]
        ws = jnp.stack(ws)
        bs = jnp.stack(bs)[:, None, :]
        if last:
            wrbf = jnp.zeros((NRBF, H), jnp.float32)
        else:
            pn = ints[l + 1]
            ws = jnp.concatenate([ws, pn["lin_kj"]["w"][None]], axis=0)
            bs = jnp.concatenate([bs, pn["lin_kj"]["b"][None, None]], axis=0)
            wrbf = pn["lin_rbf"]

        res = _tc_edge(m, aggr, rbf, (ws, bs, wrbf), last)
        if last:
            m, = res
        else:
            m, xkf = res
        owl, obl = stack_out(outs[l + 1])
        s_acc = _tc_outblock(m, s_acc, owl, obl)

    return _sc_scatter_add(s_acc, j_idx, N, [5000])


# speedup vs baseline: 2.1705x; 1.0325x over previous
import functools

import jax
import jax.numpy as jnp
from jax import lax
from jax.experimental import pallas as pl
from jax.experimental.pallas import tpu as pltpu
from jax.experimental.pallas import tpu_sc as plsc

N = 10000
E = 160000
T = 160000
H = 128
DE = 16
NRBF = 16
NSBF = 7
NB = 8
NL = 4
NOUT = 3

NC = 2
NS = 16
NW = NC * NS


def _sc_gather(table, idx, chunk):
    B = idx.shape[0]
    R, D = table.shape
    per_w = B // NW
    n_chunks = per_w // chunk
    assert per_w % chunk == 0 and chunk % 8 == 0 and B % NW == 0
    assert n_chunks % 2 == 1 and n_chunks >= 3

    mesh = plsc.VectorSubcoreMesh(core_axis_name="c", subcore_axis_name="s")

    @functools.partial(
        pl.kernel,
        mesh=mesh,
        out_type=jax.ShapeDtypeStruct((B, D), jnp.float32),
        scratch_types=[
            pltpu.VMEM((per_w,), jnp.int32),
            pltpu.VMEM((chunk, D), jnp.float32),
            pltpu.VMEM((chunk, D), jnp.float32),
            pltpu.SemaphoreType.DMA,
            pltpu.SemaphoreType.DMA,
        ],
        name=f"sc_gather_{R}x{D}_{B}",
    )
    def k(table_hbm, idx_hbm, out_hbm, idx_v, rows0, rows1, sem0, sem1):
        wid = lax.axis_index("s") * NC + lax.axis_index("c")
        base = wid * per_w
        pltpu.sync_copy(idx_hbm.at[pl.ds(base, per_w)], idx_v)

        def start(ci, rows, sem):
            pltpu.async_copy(
                table_hbm.at[idx_v.at[pl.ds(ci * chunk, chunk)]], rows, sem)

        def fin(ci, rows, sem):
            pltpu.make_async_copy(
                table_hbm.at[idx_v.at[pl.ds(0, chunk)]], rows, sem).wait()
            pltpu.sync_copy(rows, out_hbm.at[pl.ds(base + ci * chunk, chunk)])

        start(0, rows0, sem0)
        start(1, rows1, sem1)

        def body(i2, _):
            c0 = 2 * i2
            fin(c0, rows0, sem0)

            @pl.when(c0 + 2 < n_chunks)
            def _():
                start(c0 + 2, rows0, sem0)

            fin(c0 + 1, rows1, sem1)

            @pl.when(c0 + 3 < n_chunks)
            def _():
                start(c0 + 3, rows1, sem1)

            return 0

        lax.fori_loop(0, n_chunks // 2, body, 0)
        fin(n_chunks - 1, rows0, sem0)

    return k(table, idx)


def _sc_scatter_add(values, idx, R, phases):
    B, D = values.shape
    assert D == H
    per_w = B // NS
    r_half = R // NC
    assert sum(phases) == r_half
    SLAB = 80
    n_slab = per_w // SLAB
    assert per_w % SLAB == 0 and n_slab % 2 == 1 and n_slab >= 3
    max_phase = max(phases)
    ZB = 16
    n_zchunk_max = (max_phase + 80 + ZB - 1) // ZB
    acc_rows = n_zchunk_max * ZB
    ZR = 200
    assert all(p % ZR == 0 for p in phases)

    mesh = plsc.VectorSubcoreMesh(core_axis_name="c", subcore_axis_name="s")

    @functools.partial(
        pl.kernel,
        mesh=mesh,
        out_type=jax.ShapeDtypeStruct((R, D), jnp.float32),
        scratch_types=[
            pltpu.VMEM((SLAB, H), jnp.float32),
            pltpu.VMEM((SLAB, H), jnp.float32),
            pltpu.VMEM((SLAB,), jnp.int32),
            pltpu.VMEM((SLAB,), jnp.int32),
            pltpu.VMEM((1, SLAB), jnp.int32),
            pltpu.VMEM((1, SLAB), jnp.int32),
            pltpu.VMEM((ZB, H), jnp.float32),
            pltpu.VMEM((16,), jnp.int32),
            pltpu.VMEM_SHARED((acc_rows, H), jnp.float32),
            pltpu.SemaphoreType.DMA,
            pltpu.SemaphoreType.DMA,
            pltpu.SemaphoreType.DMA,
            pltpu.SemaphoreType.DMA,
        ],
        name=f"sc_scatter_{B}to{R}",
    )
    def k(vals_hbm, idx_hbm, lov_hbm, out_hbm,
          rows0, rows1, isl0, isl1, lidx0, lidx1, zero_v, lo_v, acc_sh,
          sem0, sem1, semz, sems):
        c = lax.axis_index("c")
        s = lax.axis_index("s")
        base = s * per_w
        pltpu.sync_copy(lov_hbm.at[c], lo_v)
        lo_c = lo_v[...]
        zf = jnp.zeros((16,), jnp.float32)

        iota16 = lax.iota(jnp.int32, 16)

        def zinit(i, _):
            r = i // 8
            q = i - r * 8
            zero_v[r, pl.ds(q * 16, 16)] = zf
            return 0

        lax.fori_loop(0, ZB * 8, zinit, 0)

        def start(sl, rows, isl, sem):
            pltpu.async_copy(
                vals_hbm.at[pl.ds(base + sl * SLAB, SLAB)], rows, sem)
            pltpu.async_copy(
                idx_hbm.at[pl.ds(base + sl * SLAB, SLAB)], isl, sem)

        p_lo = 0
        for p, phase_rows in enumerate(phases):
            hrow = jnp.full((16,), phase_rows, jnp.int32)
            lov = lo_c + jnp.full((16,), p_lo, jnp.int32)
            hiv = lov + hrow
            n_z = (phase_rows + 80 + ZB - 1) // ZB

            n_zw = (n_z + NS - 1) // NS
            for j in range(n_zw):
                zi = j * NS + s

                @pl.when(zi < n_z)
                def _():
                    pltpu.async_copy(zero_v, acc_sh.at[pl.ds(zi * ZB, ZB)],
                                     semz)

            for j in range(n_zw):
                zi = j * NS + s

                @pl.when(zi < n_z)
                def _():
                    pltpu.make_async_copy(
                        zero_v, acc_sh.at[pl.ds(0, ZB)], semz).wait()

            plsc.subcore_barrier()

            def fin(sl, rows, isl, lidx, sem):
                pltpu.make_async_copy(
                    vals_hbm.at[pl.ds(0, SLAB)], rows, sem).wait()
                pltpu.make_async_copy(
                    idx_hbm.at[pl.ds(0, SLAB)], isl, sem).wait()
                for q in range(SLAB // 16):
                    trash = hrow + jnp.full((16,), q * 16, jnp.int32) + iota16
                    v = isl[pl.ds(q * 16, 16)]
                    loc = lax.select(v >= lov, v - lov, trash)
                    loc = lax.select(v < hiv, loc, trash)
                    lidx[0, pl.ds(q * 16, 16)] = loc
                pltpu.sync_copy(rows, acc_sh.at[lidx.at[0]], add=True)

            start(0, rows0, isl0, sem0)
            start(1, rows1, isl1, sem1)

            def sbody(i2, _):
                sl0 = 2 * i2
                fin(sl0, rows0, isl0, lidx0, sem0)

                @pl.when(sl0 + 2 < n_slab)
                def _():
                    start(sl0 + 2, rows0, isl0, sem0)

                fin(sl0 + 1, rows1, isl1, lidx1, sem1)

                @pl.when(sl0 + 3 < n_slab)
                def _():
                    start(sl0 + 3, rows1, isl1, sem1)

                return 0

            lax.fori_loop(0, n_slab // 2, sbody, 0)
            fin(n_slab - 1, rows0, isl0, lidx0, sem0)
            plsc.subcore_barrier()

            n_wb = phase_rows // ZR
            n_ww = (n_wb + NS - 1) // NS
            for j in range(n_ww):
                zi = j * NS + s

                @pl.when(zi < n_wb)
                def _():
                    pltpu.async_copy(acc_sh.at[pl.ds(zi * ZR, ZR)],
                                     out_hbm.at[pl.ds(c * r_half + p_lo
                                                      + zi * ZR, ZR)], semz)

            for j in range(n_ww):
                zi = j * NS + s

                @pl.when(zi < n_wb)
                def _():
                    pltpu.make_async_copy(
                        acc_sh.at[pl.ds(0, ZR)],
                        out_hbm.at[pl.ds(0, ZR)], semz).wait()

            plsc.subcore_barrier()
            p_lo += phase_rows

    lov_all = (jnp.arange(NC, dtype=jnp.int32) * r_half)[:, None] + jnp.zeros(
        (NC, 16), jnp.int32)
    return k(values, idx, lov_all)


BE = 3200
BT = 3200


def _sig(v):
    return 1.0 / (1.0 + jnp.exp(-v))


def _sw(v):
    return v * _sig(v)


def _mm(a, w):
    return jnp.dot(a, w, preferred_element_type=jnp.float32)


def _res_blk(v, w1, b1, w2, b2):
    return v + _mm(_sw(_mm(_sw(v), w1) + b1), w2) + b2


def _sin_poly(x):
    x2 = x * x
    return x * (1.0 + x2 * (-1.0 / 6.0 + x2 * (1.0 / 120.0 + x2 * (-1.0 / 5040.0))))


def _cos_poly(x):
    x2 = x * x
    return 1.0 + x2 * (-0.5 + x2 * (1.0 / 24.0 + x2 * (-1.0 / 720.0 + x2 * (1.0 / 40320.0))))


def _cheb_sin_table(s1, co, n):
    cols = []
    um1 = jnp.ones_like(co)
    u = 2.0 * co
    cols.append(s1 * um1)
    for _ in range(n - 1):
        cols.append(s1 * u)
        um1, u = u, 2.0 * co * u - um1
    return cols


def _full_spec(shape):
    return pl.BlockSpec(shape, lambda i: tuple(0 for _ in shape))


def _row_spec(bs, d):
    return pl.BlockSpec((bs, d), lambda i: (i, 0))


def _tc_node_embed(x, wn, bn):
    BN = 400

    def body(x_r, w_r, b_r, o_r):
        o_r[...] = _mm(x_r[...], w_r[...]) + b_r[...]

    return pl.pallas_call(
        body,
        grid=(N // BN,),
        in_specs=[_row_spec(BN, H), _full_spec((H, H)), _full_spec((1, H))],
        out_specs=_row_spec(BN, H),
        out_shape=jax.ShapeDtypeStruct((N, H), jnp.float32),
    )(x, wn, bn.reshape(1, H))


def _tc_embed(xi_g, xj_g, ea, dist2d, wts):
    def body(xi_r, xj_r, ea_r, d_r,
             we_r, be_r, wr_r, br_r, w3_r, b3_r,
             wkj_r, bkj_r, wrbf_r,
             m_o, rbf_o, u_o, xkf_o):
        d = d_r[...]
        t0 = d * (jnp.pi / 5.0)
        s1 = _sin_poly(t0)
        co = _cos_poly(t0)
        dinv = 1.0 / (d + 1e-8)
        cols = _cheb_sin_table(s1, co, NRBF)
        rbf = jnp.concatenate(cols, axis=1) * dinv
        rbf_o[...] = rbf

        ea_v = ea_r[...]
        nrm = jnp.sqrt(jnp.sum(ea_v * ea_v, axis=1, keepdims=True))
        u = ea_v / jnp.maximum(nrm, 1e-8)
        u_o[...] = jnp.concatenate(
            [u, jnp.zeros((u.shape[0], H - DE), jnp.float32)], axis=1)

        e_lin = _mm(ea_v, we_r[...]) + be_r[...]
        r_lin = _mm(rbf, wr_r[...]) + br_r[...]
        xixj = xi_r[...] * xj_r[...]
        w3 = w3_r[...]
        m = _sw(_mm(xixj, w3[0]) + _mm(e_lin, w3[1]) + _mm(r_lin, w3[2])
                + b3_r[...])
        m_o[...] = m

        xkf_o[...] = _sw(_mm(m, wkj_r[...]) + bkj_r[...]) * _mm(rbf, wrbf_r[...])

    (we, be, wr, br, w3, b3, wkj, bkj, wrbf) = wts
    return pl.pallas_call(
        body,
        grid=(E // BE,),
        in_specs=[
            _row_spec(BE, H), _row_spec(BE, H), _row_spec(BE, DE),
            _row_spec(BE, 1),
            _full_spec((DE, H)), _full_spec((1, H)),
            _full_spec((NRBF, H)), _full_spec((1, H)),
            _full_spec((3, H, H)), _full_spec((1, H)),
            _full_spec((H, H)), _full_spec((1, H)),
            _full_spec((NRBF, H)),
        ],
        out_specs=[
            _row_spec(BE, H), _row_spec(BE, NRBF), _row_spec(BE, H),
            _row_spec(BE, H),
        ],
        out_shape=[
            jax.ShapeDtypeStruct((E, H), jnp.float32),
            jax.ShapeDtypeStruct((E, NRBF), jnp.float32),
            jax.ShapeDtypeStruct((E, H), jnp.float32),
            jax.ShapeDtypeStruct((E, H), jnp.float32),
        ],
    )(xi_g, xj_g, ea, dist2d, we, be, wr, br, w3, b3, wkj, bkj, wrbf)


def _tc_sbf(ukj_g, uji_g):
    def body(a_r, b_r, o_r):
        cos = jnp.sum(a_r[:, :DE] * b_r[:, :DE], axis=1, keepdims=True)
        c = jnp.clip(cos, -0.999, 0.999)
        s1 = jnp.sqrt(1.0 - c * c)
        cols = _cheb_sin_table(s1, c, NSBF)
        cols.append(jnp.zeros_like(c))
        o_r[...] = jnp.concatenate(cols, axis=1)

    return pl.pallas_call(
        body,
        grid=(T // BT,),
        in_specs=[_row_spec(BT, H), _row_spec(BT, H)],
        out_specs=_row_spec(BT, NB),
        out_shape=jax.ShapeDtypeStruct((T, NB), jnp.float32),
    )(ukj_g, uji_g)


def _tc_outblock(m, s_in, ow, ob):
    first = s_in is None

    def body(*refs):
        if first:
            m_r, ow_r, ob_r, s_o = refs
        else:
            m_r, s_r, ow_r, ob_r, s_o = refs
        ow_v = ow_r[...]
        ob_v = ob_r[...]
        v = m_r[...]
        for i in range(NOUT):
            v = _res_blk(v, ow_v[2 * i], ob_v[2 * i:2 * i + 1, 0],
                         ow_v[2 * i + 1], ob_v[2 * i + 1:2 * i + 2, 0])
        v = _sw(_mm(v, ow_v[6]) + ob_v[6:7, 0])
        s_o[...] = v if first else s_r[...] + v

    in_specs = [_row_spec(BE, H)]
    args = [m]
    if not first:
        in_specs.append(_row_spec(BE, H))
        args.append(s_in)
    in_specs += [_full_spec((7, H, H)), _full_spec((7, 1, H))]
    args += [ow, ob]
    return pl.pallas_call(
        body,
        grid=(E // BE,),
        in_specs=in_specs,
        out_specs=_row_spec(BE, H),
        out_shape=jax.ShapeDtypeStruct((E, H), jnp.float32),
    )(*args)


def _tc_triplet(xk_g, sbf, wflat, lsb):
    def body(xk_r, sbf_r, wf_r, ls_r, o_r):
        sp = _mm(sbf_r[...], ls_r[...])
        xk = xk_r[...].astype(jnp.bfloat16)
        wf = wf_r[...]
        acc = sp[:, 0:1] * _mm(xk, wf[0])
        for b in range(1, NB):
            acc = acc + sp[:, b:b + 1] * _mm(xk, wf[b])
        o_r[...] = acc

    return pl.pallas_call(
        body,
        grid=(T // BT,),
        in_specs=[
            _row_spec(BT, H), _row_spec(BT, NB),
            _full_spec((NB, H, H)), _full_spec((NB, NB)),
        ],
        out_specs=_row_spec(BT, H),
        out_shape=jax.ShapeDtypeStruct((T, H), jnp.float32),
    )(xk_g, sbf, wflat, lsb)


def _tc_edge(m, aggr, rbf, wts, last):
    def body(m_r, ag_r, rbf_r, ws_r, bs_r, wrbf_r, *out_refs):
        ws = ws_r[...]
        bs = bs_r[...]
        m_v = m_r[...]
        mm = _sw(_mm(m_v, ws[0]) + bs[0:1, 0]) + ag_r[...]
        mm = _res_blk(mm, ws[1], bs[1:2, 0], ws[2], bs[2:3, 0])
        mm = _res_blk(mm, ws[3], bs[3:4, 0], ws[4], bs[4:5, 0])
        m_new = m_v + _sw(_mm(mm, ws[5]) + bs[5:6, 0])

        if last:
            m_o, = out_refs
            m_o[...] = m_new
        else:
            m_o, xkf_o = out_refs
            m_o[...] = m_new
            xkf_o[...] = (_sw(_mm(m_new, ws[6]) + bs[6:7, 0])
                          * _mm(rbf_r[...], wrbf_r[...]))

    ws, bs, wrbf = wts
    nw = ws.shape[0]
    nout_arrs = 1 if last else 2
    return pl.pallas_call(
        body,
        grid=(E // BE,),
        in_specs=[
            _row_spec(BE, H), _row_spec(BE, H), _row_spec(BE, NRBF),
            _full_spec((nw, H, H)), _full_spec((nw, 1, H)),
            _full_spec((NRBF, H)),
        ],
        out_specs=[_row_spec(BE, H)] * nout_arrs,
        out_shape=[jax.ShapeDtypeStruct((E, H), jnp.float32)] * nout_arrs,
    )(m, aggr, rbf, ws, bs, wrbf)


def kernel(x, edge_attr, dist, edge_index, triplet_index, params):
    emb = params["emb"]
    ints = params["int"]
    outs = params["out"]

    i_idx = edge_index[0].astype(jnp.int32)
    j_idx = edge_index[1].astype(jnp.int32)
    kj = triplet_index[0].astype(jnp.int32)
    ji = triplet_index[1].astype(jnp.int32)

    xemb = _tc_node_embed(x, emb["node"]["w"], emb["node"]["b"])
    xi_g = _sc_gather(xemb, i_idx, 200)
    xj_g = _sc_gather(xemb, j_idx, 200)

    def stack_out(p):
        ws = []
        bs = []
        for rp in p["res"]:
            ws += [rp["lin1"]["w"], rp["lin2"]["w"]]
            bs += [rp["lin1"]["b"], rp["lin2"]["b"]]
        ws.append(p["lin"]["w"])
        bs.append(p["lin"]["b"])
        return jnp.stack(ws), jnp.stack(bs)[:, None, :]

    l1 = ints[0]
    wts0 = (emb["edge"]["w"], emb["edge"]["b"].reshape(1, H),
            emb["rbf"]["w"], emb["rbf"]["b"].reshape(1, H),
            jnp.stack([emb["lin"]["w"][0:H], emb["lin"]["w"][H:2 * H],
                       emb["lin"]["w"][2 * H:3 * H]]),
            emb["lin"]["b"].reshape(1, H),
            l1["lin_kj"]["w"], l1["lin_kj"]["b"].reshape(1, H),
            l1["lin_rbf"])
    m, rbf, u, xkf = _tc_embed(
        xi_g, xj_g, edge_attr, dist.reshape(E, 1), wts0)
    ow0, ob0 = stack_out(outs[0])
    s_acc = _tc_outblock(m, None, ow0, ob0)

    ukj_g = _sc_gather(u, kj, 200)
    uji_g = _sc_gather(u, ji, 200)
    sbf = _tc_sbf(ukj_g, uji_g)

    for l in range(NL):
        p = ints[l]
        last = l == NL - 1

        xk_g = _sc_gather(xkf, kj, 200)
        wflat = jnp.transpose(p["W"], (1, 0, 2)).astype(jnp.bfloat16)
        lsb = jnp.concatenate(
            [p["lin_sbf"], jnp.zeros((1, NB), jnp.float32)], axis=0)
        msg = _tc_triplet(xk_g, sbf, wflat, lsb)
        aggr = _sc_scatter_add(msg, ji, E, [13400] * 5 + [13000])

        ws = [p["lin_ji"]["w"],
              p["res1"]["lin1"]["w"], p["res1"]["lin2"]["w"],
              p["res2"]["lin1"]["w"], p["res2"]["lin2"]["w"],
              p["lin_out"]["w"]]
        bs = [p["lin_ji"]["b"],
              p["res1"]["lin1"]["b"], p["res1"]["lin2"]["b"],
              p["res2"]["lin1"]["b"], p["res2"]["lin2"]["b"],
              p["lin_out"]["b"]]
        ws = jnp.stack(ws)
        bs = jnp.stack(bs)[:, None, :]
        if last:
            wrbf = jnp.zeros((NRBF, H), jnp.float32)
        else:
            pn = ints[l + 1]
            ws = jnp.concatenate([ws, pn["lin_kj"]["w"][None]], axis=0)
            bs = jnp.concatenate([bs, pn["lin_kj"]["b"][None, None]], axis=0)
            wrbf = pn["lin_rbf"]

        res = _tc_edge(m, aggr, rbf, (ws, bs, wrbf), last)
        if last:
            m, = res
        else:
            m, xkf = res
        owl, obl = stack_out(outs[l + 1])
        s_acc = _tc_outblock(m, s_acc, owl, obl)

    return _sc_scatter_add(s_acc, j_idx, N, [5000])
